# baseline (speedup 1.0000x reference)
import functools

import jax
import jax.numpy as jnp
from jax import lax
from jax.experimental import pallas as pl
from jax.experimental.pallas import tpu as pltpu
from jax.experimental.pallas import tpu_sc as plsc

N_USERS = 1000000
N_ITEMS = 1000000
D = 64
BATCH = 16384

NC = 2
NS = 16
L = 16
NW = NC * NS
BPW = BATCH // NW
CHUNK = 128
NCHUNK = BPW // CHUNK


def _mf_body(user_h, pos_h, neg_h, umat_h, imat_h, out_h,
             idx_u, idx_p, idx_n, rows_u, rows_p, rows_n, out_v, sem):
    c = lax.axis_index("c")
    s = lax.axis_index("s")
    wid = s * NC + c
    base = wid * BPW

    for k in range(NCHUNK):
        off = base + k * CHUNK
        pltpu.sync_copy(user_h.at[pl.ds(off, CHUNK)], idx_u.at[k])
        pltpu.sync_copy(pos_h.at[pl.ds(off, CHUNK)], idx_p.at[k])
        pltpu.sync_copy(neg_h.at[pl.ds(off, CHUNK)], idx_n.at[k])

    copies = []
    for k in range(NCHUNK):
        dst = pl.ds(k * CHUNK, CHUNK)
        copies.append(pltpu.async_copy(umat_h.at[idx_u.at[k]], rows_u.at[dst], sem))
        copies.append(pltpu.async_copy(imat_h.at[idx_p.at[k]], rows_p.at[dst], sem))
        copies.append(pltpu.async_copy(imat_h.at[idx_n.at[k]], rows_n.at[dst], sem))
    for cp in copies:
        cp.wait()

    iot = lax.iota(jnp.int32, L)

    def group_body(g, carry):
        rbase = g * L
        row_idx = rbase + iot
        acc = jnp.zeros((L,), jnp.float32)
        for d in range(D):
            cd = jnp.full((L,), d, jnp.int32)
            uv = plsc.load_gather(rows_u, [row_idx, cd])
            iv = plsc.load_gather(rows_p, [row_idx, cd])
            jv = plsc.load_gather(rows_n, [row_idx, cd])
            acc = acc + uv * (iv - jv)
        out_v[pl.ds(rbase, L)] = acc
        return carry

    lax.fori_loop(0, BPW // L, group_body, 0)

    pltpu.sync_copy(out_v, out_h.at[pl.ds(base, BPW)])


@jax.jit
def _mf(user, pos, neg, user_mat, item_mat):
    mesh = plsc.VectorSubcoreMesh(core_axis_name="c", subcore_axis_name="s")
    kfn = functools.partial(
        pl.kernel,
        out_type=jax.ShapeDtypeStruct((BATCH,), jnp.float32),
        mesh=mesh,
        compiler_params=pltpu.CompilerParams(
            needs_layout_passes=False, use_tc_tiling_on_sc=False),
        scratch_types=[
            pltpu.VMEM((NCHUNK, CHUNK), jnp.int32),
            pltpu.VMEM((NCHUNK, CHUNK), jnp.int32),
            pltpu.VMEM((NCHUNK, CHUNK), jnp.int32),
            pltpu.VMEM((BPW, D), jnp.float32),
            pltpu.VMEM((BPW, D), jnp.float32),
            pltpu.VMEM((BPW, D), jnp.float32),
            pltpu.VMEM((BPW,), jnp.float32),
            pltpu.SemaphoreType.DMA,
        ],
    )(_mf_body)
    return kfn(user, pos, neg, user_mat, item_mat)


def kernel(user, pos, neg, user_mat, item_mat):
    user = user.astype(jnp.int32)
    pos = pos.astype(jnp.int32)
    neg = neg.astype(jnp.int32)
    return _mf(user, pos, neg, user_mat, item_mat)


# speedup vs baseline: 1.0129x; 1.0129x over previous
import functools

import jax
import jax.numpy as jnp
from jax import lax
from jax.experimental import pallas as pl
from jax.experimental.pallas import tpu as pltpu
from jax.experimental.pallas import tpu_sc as plsc

D = 64
DP = 128
BATCH = 16384

NC = 2
NS = 16
L = 16
NW = NC * NS
BPW = BATCH // NW
CHUNK = 128
NCHUNK = BPW // CHUNK
NBUF = 2


def _mf_body(user_h, pos_h, neg_h, umat_h, imat_h, out_h,
             idx_u, idx_p, idx_n, pid_u, pid_p, pid_n,
             rows_u, rows_p, rows_n, out_v, sem0, sem1):
    c = lax.axis_index("c")
    s = lax.axis_index("s")
    wid = s * NC + c
    base = wid * BPW
    sems = (sem0, sem1)

    for k in range(NCHUNK):
        off = base + k * CHUNK
        pltpu.sync_copy(user_h.at[pl.ds(off, CHUNK)], idx_u.at[k])
        pltpu.sync_copy(pos_h.at[pl.ds(off, CHUNK)], idx_p.at[k])
        pltpu.sync_copy(neg_h.at[pl.ds(off, CHUNK)], idx_n.at[k])
    for k in range(NCHUNK):
        for v in range(CHUNK // L):
            sl = pl.ds(v * L, L)
            pid_u[k, sl] = idx_u[k, sl] >> 1
            pid_p[k, sl] = idx_p[k, sl] >> 1
            pid_n[k, sl] = idx_n[k, sl] >> 1

    iot = lax.iota(jnp.int32, L)

    def fire(k):
        b = k % NBUF
        sem = sems[b]
        return (
            pltpu.async_copy(umat_h.at[pid_u.at[k]], rows_u.at[b], sem),
            pltpu.async_copy(imat_h.at[pid_p.at[k]], rows_p.at[b], sem),
            pltpu.async_copy(imat_h.at[pid_n.at[k]], rows_n.at[b], sem),
        )

    def compute(k):
        b = k % NBUF
        ru, rp, rn = rows_u.at[b], rows_p.at[b], rows_n.at[b]

        def group_body(g, carry):
            rbase = g * L
            rows16 = rbase + iot
            sl = pl.ds(rbase, L)
            hu = (idx_u[k, sl] & 1) * D
            hp = (idx_p[k, sl] & 1) * D
            hn = (idx_n[k, sl] & 1) * D
            acc = jnp.zeros((L,), jnp.float32)
            for d in range(D):
                uv = plsc.load_gather(ru, [rows16, hu + d])
                iv = plsc.load_gather(rp, [rows16, hp + d])
                jv = plsc.load_gather(rn, [rows16, hn + d])
                acc = acc + uv * (iv - jv)
            out_v[pl.ds(k * CHUNK + rbase, L)] = acc
            return carry

        lax.fori_loop(0, CHUNK // L, group_body, 0)

    pending = {}
    for k in range(NCHUNK + 1):
        if k < NCHUNK:
            pending[k] = fire(k)
        if k >= 1:
            for cp in pending.pop(k - 1):
                cp.wait()
            compute(k - 1)

    pltpu.sync_copy(out_v, out_h.at[pl.ds(base, BPW)])


@jax.jit
def _mf(user, pos, neg, umat2, imat2):
    mesh = plsc.VectorSubcoreMesh(core_axis_name="c", subcore_axis_name="s")
    kfn = functools.partial(
        pl.kernel,
        out_type=jax.ShapeDtypeStruct((BATCH,), jnp.float32),
        mesh=mesh,
        compiler_params=pltpu.CompilerParams(
            needs_layout_passes=False, use_tc_tiling_on_sc=False),
        scratch_types=[
            pltpu.VMEM((NCHUNK, CHUNK), jnp.int32),
            pltpu.VMEM((NCHUNK, CHUNK), jnp.int32),
            pltpu.VMEM((NCHUNK, CHUNK), jnp.int32),
            pltpu.VMEM((NCHUNK, CHUNK), jnp.int32),
            pltpu.VMEM((NCHUNK, CHUNK), jnp.int32),
            pltpu.VMEM((NCHUNK, CHUNK), jnp.int32),
            pltpu.VMEM((NBUF, CHUNK, DP), jnp.float32),
            pltpu.VMEM((NBUF, CHUNK, DP), jnp.float32),
            pltpu.VMEM((NBUF, CHUNK, DP), jnp.float32),
            pltpu.VMEM((BPW,), jnp.float32),
            pltpu.SemaphoreType.DMA,
            pltpu.SemaphoreType.DMA,
        ],
    )(_mf_body)
    return kfn(user, pos, neg, umat2, imat2)


def kernel(user, pos, neg, user_mat, item_mat):
    user = user.astype(jnp.int32)
    pos = pos.astype(jnp.int32)
    neg = neg.astype(jnp.int32)
    umat2 = user_mat.reshape(user_mat.shape[0] // 2, DP)
    imat2 = item_mat.reshape(item_mat.shape[0] // 2, DP)
    return _mf(user, pos, neg, umat2, imat2)
